# baseline (speedup 1.0000x reference)
import functools

import jax
import jax.numpy as jnp
from jax import lax
from jax.experimental import pallas as pl
from jax.experimental.pallas import tpu as pltpu, tpu_sc as plsc

NC = 2
NS = 16
L = 16
D = 128
CHUNK = 64
NSLOT = 6
UNROLL = 2


def _rsqrt(x):
    i = plsc.bitcast(x, jnp.int32)
    i = jnp.int32(0x5F3759DF) - (i >> 1)
    y = plsc.bitcast(i, jnp.float32)
    for _ in range(3):
        y = y * (1.5 - 0.5 * x * y * y)
    return y


def _make_kernel(B, V):
    n_workers = NC * NS
    per_w = B // n_workers
    n_chunks = per_w // CHUNK
    mesh = plsc.VectorSubcoreMesh(core_axis_name="c", subcore_axis_name="s")

    @functools.partial(
        pl.kernel,
        out_type=jax.ShapeDtypeStruct((B,), jnp.float32),
        mesh=mesh,
        compiler_params=pltpu.CompilerParams(needs_layout_passes=False, use_tc_tiling_on_sc=False),
        scratch_types=[
            pltpu.VMEM((per_w,), jnp.int32),
            pltpu.VMEM((per_w,), jnp.int32),
            pltpu.VMEM((NSLOT, CHUNK, D), jnp.float32),
            pltpu.VMEM((NSLOT, CHUNK, D), jnp.float32),
            pltpu.VMEM((2, L), jnp.float32),
            pltpu.VMEM((per_w + L,), jnp.float32),
            pltpu.VMEM((per_w + L,), jnp.float32),
            pltpu.VMEM((per_w + L,), jnp.float32),
            pltpu.VMEM((per_w,), jnp.float32),
        ] + [pltpu.SemaphoreType.DMA] * NSLOT,
    )
    def _k(words_hbm, ctx_hbm, table_hbm, wb_hbm, out_hbm,
           widx, cidx, webuf, cebuf, wb_v,
           dotbuf, w2buf, c2buf, outbuf, *sems):
        wid = lax.axis_index("s") * NC + lax.axis_index("c")
        base = wid * per_w

        pltpu.sync_copy(words_hbm.at[pl.ds(base, per_w)], widx)
        pltpu.sync_copy(ctx_hbm.at[pl.ds(base, per_w)], cidx)
        pltpu.sync_copy(wb_hbm, wb_v)

        def start(c):
            slot = c % NSLOT
            return (
                pltpu.async_copy(table_hbm.at[widx.at[pl.ds(c * CHUNK, CHUNK)]],
                                 webuf.at[slot], sems[slot]),
                pltpu.async_copy(table_hbm.at[cidx.at[pl.ds(c * CHUNK, CHUNK)]],
                                 cebuf.at[slot], sems[slot]),
            )

        inflight = [start(c) for c in range(NSLOT - 1)]
        for c in range(n_chunks):
            if c + NSLOT - 1 < n_chunks:
                inflight.append(start(c + NSLOT - 1))
            for h in inflight.pop(0):
                h.wait()
            we_b = webuf.at[c % NSLOT]
            ce_b = cebuf.at[c % NSLOT]

            def pair_body(p):
                dotv = jnp.zeros((L,), jnp.float32)
                w2v = jnp.zeros((L,), jnp.float32)
                c2v = jnp.zeros((L,), jnp.float32)
                for k in range(D // L):
                    we = we_b[p, pl.ds(k * L, L)]
                    ce = ce_b[p, pl.ds(k * L, L)]
                    dotv += we * ce
                    w2v += we * we
                    c2v += ce * ce
                m15 = lax.iota(jnp.int32, L) == (L - 1)
                q = c * CHUNK + p
                plsc.store_compressed(dotbuf.at[pl.ds(q, L)],
                                      plsc.cumsum(dotv), mask=m15)
                plsc.store_compressed(w2buf.at[pl.ds(q, L)],
                                      plsc.cumsum(w2v), mask=m15)
                plsc.store_compressed(c2buf.at[pl.ds(q, L)],
                                      plsc.cumsum(c2v), mask=m15)

            plsc.parallel_loop(0, CHUNK, unroll=UNROLL)(pair_body)

        wv = wb_v[0, pl.ds(0, L)]
        bv = wb_v[1, pl.ds(0, L)]

        def group_body(g):
            dot = dotbuf[pl.ds(g * L, L)]
            w2 = w2buf[pl.ds(g * L, L)]
            c2 = c2buf[pl.ds(g * L, L)]
            sim = dot * _rsqrt(jnp.maximum(w2, 1e-16)) \
                      * _rsqrt(jnp.maximum(c2, 1e-16))
            logit = sim * wv + bv
            outbuf[pl.ds(g * L, L)] = 1.0 / (1.0 + jnp.exp(-logit))

        plsc.parallel_loop(0, per_w // L, unroll=2)(group_body)

        pltpu.sync_copy(outbuf, out_hbm.at[pl.ds(base, per_w)])

    return _k


def kernel(words, contexts, table, W, b):
    B = words.shape[0]
    V, d = table.shape
    assert d == D and B % (NC * NS * CHUNK) == 0
    wb = jnp.stack([jnp.full((L,), W[0, 0], jnp.float32),
                    jnp.full((L,), b[0], jnp.float32)])
    if words.dtype != jnp.int32:
        words = words.astype(jnp.int32)
    if contexts.dtype != jnp.int32:
        contexts = contexts.astype(jnp.int32)
    out = _make_kernel(B, V)(words, contexts, table, wb)
    return out.reshape(B, 1, 1)


# speedup vs baseline: 1.0237x; 1.0237x over previous
import functools

import jax
import jax.numpy as jnp
from jax import lax
from jax.experimental import pallas as pl
from jax.experimental.pallas import tpu as pltpu, tpu_sc as plsc

NC = 2
NS = 16
L = 16
D = 128
CHUNK = 128
NSLOT = 3
UNROLL = 2


def _rsqrt(x):
    i = plsc.bitcast(x, jnp.int32)
    i = jnp.int32(0x5F3759DF) - (i >> 1)
    y = plsc.bitcast(i, jnp.float32)
    for _ in range(3):
        y = y * (1.5 - 0.5 * x * y * y)
    return y


def _make_kernel(B, V):
    n_workers = NC * NS
    per_w = B // n_workers
    n_chunks = per_w // CHUNK
    mesh = plsc.VectorSubcoreMesh(core_axis_name="c", subcore_axis_name="s")

    @functools.partial(
        pl.kernel,
        out_type=jax.ShapeDtypeStruct((B,), jnp.float32),
        mesh=mesh,
        compiler_params=pltpu.CompilerParams(needs_layout_passes=False, use_tc_tiling_on_sc=False),
        scratch_types=[
            pltpu.VMEM((per_w,), jnp.int32),
            pltpu.VMEM((per_w,), jnp.int32),
            pltpu.VMEM((NSLOT, CHUNK, D), jnp.float32),
            pltpu.VMEM((NSLOT, CHUNK, D), jnp.float32),
            pltpu.VMEM((2, L), jnp.float32),
            pltpu.VMEM((per_w + L,), jnp.float32),
            pltpu.VMEM((per_w + L,), jnp.float32),
            pltpu.VMEM((per_w + L,), jnp.float32),
            pltpu.VMEM((per_w,), jnp.float32),
        ] + [pltpu.SemaphoreType.DMA] * NSLOT,
    )
    def _k(words_hbm, ctx_hbm, table_hbm, wb_hbm, out_hbm,
           widx, cidx, webuf, cebuf, wb_v,
           dotbuf, w2buf, c2buf, outbuf, *sems):
        wid = lax.axis_index("s") * NC + lax.axis_index("c")
        base = wid * per_w

        pltpu.sync_copy(words_hbm.at[pl.ds(base, per_w)], widx)
        pltpu.sync_copy(ctx_hbm.at[pl.ds(base, per_w)], cidx)
        pltpu.sync_copy(wb_hbm, wb_v)

        def start(c):
            slot = c % NSLOT
            return (
                pltpu.async_copy(table_hbm.at[widx.at[pl.ds(c * CHUNK, CHUNK)]],
                                 webuf.at[slot], sems[slot]),
                pltpu.async_copy(table_hbm.at[cidx.at[pl.ds(c * CHUNK, CHUNK)]],
                                 cebuf.at[slot], sems[slot]),
            )

        inflight = [start(c) for c in range(NSLOT - 1)]
        for c in range(n_chunks):
            if c + NSLOT - 1 < n_chunks:
                inflight.append(start(c + NSLOT - 1))
            for h in inflight.pop(0):
                h.wait()
            we_b = webuf.at[c % NSLOT]
            ce_b = cebuf.at[c % NSLOT]

            def pair_body(p):
                dotv = jnp.zeros((L,), jnp.float32)
                w2v = jnp.zeros((L,), jnp.float32)
                c2v = jnp.zeros((L,), jnp.float32)
                for k in range(D // L):
                    we = we_b[p, pl.ds(k * L, L)]
                    ce = ce_b[p, pl.ds(k * L, L)]
                    dotv += we * ce
                    w2v += we * we
                    c2v += ce * ce
                m15 = lax.iota(jnp.int32, L) == (L - 1)
                q = c * CHUNK + p
                plsc.store_compressed(dotbuf.at[pl.ds(q, L)],
                                      plsc.cumsum(dotv), mask=m15)
                plsc.store_compressed(w2buf.at[pl.ds(q, L)],
                                      plsc.cumsum(w2v), mask=m15)
                plsc.store_compressed(c2buf.at[pl.ds(q, L)],
                                      plsc.cumsum(c2v), mask=m15)

            plsc.parallel_loop(0, CHUNK, unroll=UNROLL)(pair_body)

        wv = wb_v[0, pl.ds(0, L)]
        bv = wb_v[1, pl.ds(0, L)]

        def group_body(g):
            dot = dotbuf[pl.ds(g * L, L)]
            w2 = w2buf[pl.ds(g * L, L)]
            c2 = c2buf[pl.ds(g * L, L)]
            sim = dot * _rsqrt(jnp.maximum(w2, 1e-16)) \
                      * _rsqrt(jnp.maximum(c2, 1e-16))
            logit = sim * wv + bv
            outbuf[pl.ds(g * L, L)] = 1.0 / (1.0 + jnp.exp(-logit))

        plsc.parallel_loop(0, per_w // L, unroll=2)(group_body)

        pltpu.sync_copy(outbuf, out_hbm.at[pl.ds(base, per_w)])

    return _k


def kernel(words, contexts, table, W, b):
    B = words.shape[0]
    V, d = table.shape
    assert d == D and B % (NC * NS * CHUNK) == 0
    wb = jnp.stack([jnp.full((L,), W[0, 0], jnp.float32),
                    jnp.full((L,), b[0], jnp.float32)])
    if words.dtype != jnp.int32:
        words = words.astype(jnp.int32)
    if contexts.dtype != jnp.int32:
        contexts = contexts.astype(jnp.int32)
    out = _make_kernel(B, V)(words, contexts, table, wb)
    return out.reshape(B, 1, 1)
